# baseline (speedup 1.0000x reference)
import functools

import jax
import jax.numpy as jnp
from jax import lax
from jax.experimental import pallas as pl
from jax.experimental.pallas import tpu as pltpu
from jax.experimental.pallas import tpu_sc as plsc

N = 10000
E = 320000
D = 128

NC = 2
NS = 16
C = 128
NCHUNK = E // C
CH_PER_CORE = NCHUNK // NC
CPW = 80
CPW_LAST = NCHUNK - CPW * (NC * NS - 1)
NSLOT = 4
DRPT = N // NS
DEG_NWB = 10
DEG_WBR = N // DEG_NWB
DW = 16

_mesh = plsc.VectorSubcoreMesh(
    core_axis_name="c", subcore_axis_name="s", num_cores=NC, num_subcores=NS)


def _zero_vmem_2d(ref, rows, width):
  z16 = jnp.zeros((16,), jnp.float32)

  def row(i, _):
    for k in range(width // 16):
      ref[i, pl.ds(k * 16, 16)] = z16
    return 0

  lax.fori_loop(0, rows, row, 0)


def _zero_spmem_rows(acc, zb, base, nrows):
  zrows = zb.shape[0]
  full = nrows // zrows
  rem = nrows - full * zrows
  for k in range(full):
    pltpu.sync_copy(zb, acc.at[pl.ds(base + k * zrows, zrows)])
  if rem:
    pltpu.sync_copy(zb.at[pl.ds(0, rem)],
                    acc.at[pl.ds(base + full * zrows, rem)])


@functools.partial(
    pl.kernel,
    out_type=jax.ShapeDtypeStruct((NC, N, DW), jnp.float32),
    mesh=_mesh,
    compiler_params=pltpu.CompilerParams(use_tc_tiling_on_sc=False),
    scratch_types=[
        pltpu.VMEM_SHARED((N, DW), jnp.float32),
        pltpu.VMEM((C, DW), jnp.float32),
        pltpu.VMEM((C, DW), jnp.float32),
        pltpu.VMEM((NSLOT, C), jnp.int32),
        pltpu.SemaphoreType.DMA,
        pltpu.SemaphoreType.DMA,
        pltpu.SemaphoreType.DMA,
        pltpu.SemaphoreType.DMA,
        pltpu.SemaphoreType.DMA,
        pltpu.SemaphoreType.DMA,
        pltpu.SemaphoreType.DMA,
        pltpu.SemaphoreType.DMA,
    ],
)
def _deg_kernel(dst2d, out, degacc, ones, zb, didx,
                i0, i1, i2, i3, s0, s1, s2, s3):
  c = lax.axis_index("c")
  s = lax.axis_index("s")
  w = c * NS + s
  start = w * CPW
  cnt = jnp.where(w == NC * NS - 1, CPW_LAST, CPW)

  isems = (i0, i1, i2, i3)
  ssems = (s0, s1, s2, s3)

  def istart(j, k):
    pltpu.async_copy(dst2d.at[start + j], didx.at[k], isems[k])

  def iwait(j, k):
    pltpu.make_async_copy(dst2d.at[start + j], didx.at[k], isems[k]).wait()

  def sstart(k):
    pltpu.async_copy(ones, degacc.at[didx.at[k]], ssems[k], add=True)

  def swait(k):
    pltpu.make_async_copy(ones, degacc.at[didx.at[k]], ssems[k]).wait()

  for k in range(NSLOT):
    istart(k, k)

  one16 = jnp.ones((16,), jnp.float32)
  z16 = jnp.zeros((16,), jnp.float32)

  def fill(i, _):
    ones[i, pl.ds(0, 16)] = one16
    zb[i, pl.ds(0, 16)] = z16
    return 0

  lax.fori_loop(0, C, fill, 0)

  @pl.when(s < DEG_NWB)
  def _():
    _zero_spmem_rows(degacc, zb, s * DEG_WBR, DEG_WBR)

  plsc.subcore_barrier()

  def body(i, _):
    q = 4 * i
    for k in range(NSLOT):
      iwait(q + k, k)
      sstart(k)
    for k in range(NSLOT):
      @pl.when(q + k + NSLOT < cnt)
      def _():
        swait(k)
        istart(q + k + NSLOT, k)
    return 0

  lax.fori_loop(0, cnt // 4, body, 0)
  for k in range(NSLOT):
    swait(k)

  plsc.subcore_barrier()

  @pl.when(s < DEG_NWB)
  def _():
    pltpu.sync_copy(degacc.at[pl.ds(s * DEG_WBR, DEG_WBR)],
                    out.at[c, pl.ds(s * DEG_WBR, DEG_WBR)])


@functools.partial(
    pl.kernel,
    out_type=jax.ShapeDtypeStruct((NC, N, D), jnp.float32),
    mesh=_mesh,
    compiler_params=pltpu.CompilerParams(use_tc_tiling_on_sc=False),
    scratch_types=[
        pltpu.VMEM_SHARED((N, D), jnp.float32),
        pltpu.VMEM((2, C, D), jnp.float32),
        pltpu.VMEM((NSLOT, C), jnp.int32),
        pltpu.VMEM((NSLOT, C), jnp.int32),
        pltpu.SemaphoreType.DMA,
        pltpu.SemaphoreType.DMA,
        pltpu.SemaphoreType.DMA,
        pltpu.SemaphoreType.DMA,
        pltpu.SemaphoreType.DMA,
        pltpu.SemaphoreType.DMA,
        pltpu.SemaphoreType.DMA,
        pltpu.SemaphoreType.DMA,
    ],
)
def _scatter_kernel(g, src2d, dst2d, out, acc, rows, sidx, didx,
                    isem0, isem1, isem2, isem3, gsem0, gsem1, ssem0, ssem1):
  c = lax.axis_index("c")
  s = lax.axis_index("s")
  w = c * NS + s
  start = w * CPW
  cnt = jnp.where(w == NC * NS - 1, CPW_LAST, CPW)

  isems = (isem0, isem1, isem2, isem3)
  gsems = (gsem0, gsem1)
  ssems = (ssem0, ssem1)

  def istart(j, k):
    pltpu.async_copy(src2d.at[start + j], sidx.at[k], isems[k])
    pltpu.async_copy(dst2d.at[start + j], didx.at[k], isems[k])

  def iwait(j, k):
    pltpu.make_async_copy(src2d.at[start + j], sidx.at[k], isems[k]).wait()
    pltpu.make_async_copy(dst2d.at[start + j], didx.at[k], isems[k]).wait()

  def gstart(k, b):
    pltpu.async_copy(g.at[sidx.at[k]], rows.at[b], gsems[b])

  def gwait(k, b):
    pltpu.make_async_copy(g.at[sidx.at[k]], rows.at[b], gsems[b]).wait()

  def sstart(k, b):
    pltpu.async_copy(rows.at[b], acc.at[didx.at[k]], ssems[b], add=True)

  def swait(b):
    pltpu.make_async_copy(rows.at[b], acc.at[didx.at[b % NSLOT]],
                          ssems[b]).wait()

  for k in range(NSLOT - 1):
    istart(k, k)

  z16 = jnp.zeros((16,), jnp.float32)

  def zrow(i, _):
    for k in range(D // 16):
      rows[0, i, pl.ds(k * 16, 16)] = z16
    return 0

  lax.fori_loop(0, C, zrow, 0)

  _zero_spmem_rows(acc, rows.at[0], s * DRPT, DRPT)

  plsc.subcore_barrier()

  iwait(0, 0)
  gstart(0, 0)

  def pos(j, k, b):
    gwait(k, b)
    sstart(k, b)

    @pl.when(j + 1 < cnt)
    def _():
      @pl.when(j >= 1)
      def _():
        swait(1 - b)

      @pl.when(j + 3 < cnt)
      def _():
        istart(j + 3, (k + 3) % NSLOT)

      iwait(j + 1, (k + 1) % NSLOT)
      gstart((k + 1) % NSLOT, 1 - b)

  def body(i, _):
    q = 4 * i
    for k in range(4):
      pos(q + k, k, k % 2)
    return 0

  lax.fori_loop(0, cnt // 4, body, 0)
  swait(0)
  swait(1)

  plsc.subcore_barrier()

  pltpu.sync_copy(acc.at[pl.ds(s * DRPT, DRPT)],
                  out.at[c, pl.ds(s * DRPT, DRPT)])


_RB = 2000


def _dinv_block(da_ref, db_ref):
  deg = da_ref[:, 0:1] + db_ref[:, 0:1] + 1.0
  return lax.rsqrt(deg)


def _k1_body(x_ref, w_ref, da_ref, db_ref, o_ref):
  dinv = _dinv_block(da_ref, db_ref)
  h = jnp.dot(x_ref[...], w_ref[...], preferred_element_type=jnp.float32)
  o_ref[...] = h * dinv


def _k2_body(a0_ref, a1_ref, g_ref, b_ref, w_ref, da_ref, db_ref, o_ref):
  dinv = _dinv_block(da_ref, db_ref)
  t = (a0_ref[...] + a1_ref[...] + g_ref[...]) * dinv + b_ref[...]
  z = jnp.maximum(t, 0.0)
  h = jnp.dot(z, w_ref[...], preferred_element_type=jnp.float32)
  o_ref[...] = h * dinv


def _k3_body(a0_ref, a1_ref, g_ref, b_ref, da_ref, db_ref, o_ref):
  dinv = _dinv_block(da_ref, db_ref)
  o_ref[...] = (a0_ref[...] + a1_ref[...] + g_ref[...]) * dinv + b_ref[...]


def _row_spec(width):
  return pl.BlockSpec((_RB, width), lambda i: (i, 0))


def _full_spec(shape):
  return pl.BlockSpec(shape, lambda i: (0,) * len(shape))


def _tc_call(body, in_specs, n_out_width=D):
  return pl.pallas_call(
      body,
      grid=(N // _RB,),
      in_specs=in_specs,
      out_specs=_row_spec(n_out_width),
      out_shape=jax.ShapeDtypeStruct((N, n_out_width), jnp.float32),
  )


def kernel(x, edge_index, W1, b1, W2, b2):
  src2d = edge_index[0].reshape(NCHUNK, C)
  dst2d = edge_index[1].reshape(NCHUNK, C)

  degp = _deg_kernel(dst2d)
  dega, degb = degp[0], degp[1]

  g1 = _tc_call(
      _k1_body,
      [_row_spec(D), _full_spec((D, D)), _row_spec(DW), _row_spec(DW)],
  )(x, W1, dega, degb)

  s1 = _scatter_kernel(g1, src2d, dst2d)

  b1r = b1.reshape(1, D)
  b2r = b2.reshape(1, D)

  g2 = _tc_call(
      _k2_body,
      [_row_spec(D), _row_spec(D), _row_spec(D), _full_spec((1, D)),
       _full_spec((D, D)), _row_spec(DW), _row_spec(DW)],
  )(s1[0], s1[1], g1, b1r, W2, dega, degb)

  s2 = _scatter_kernel(g2, src2d, dst2d)

  out = _tc_call(
      _k3_body,
      [_row_spec(D), _row_spec(D), _row_spec(D), _full_spec((1, D)),
       _row_spec(DW), _row_spec(DW)],
  )(s2[0], s2[1], g2, b2r, dega, degb)

  return out


# speedup vs baseline: 1.2409x; 1.2409x over previous
import functools

import jax
import jax.numpy as jnp
from jax import lax
from jax.experimental import pallas as pl
from jax.experimental.pallas import tpu as pltpu
from jax.experimental.pallas import tpu_sc as plsc

N = 10000
E = 320000
D = 128

NC = 2
NS = 16
C = 128
NCHUNK = E // C
CH_PER_CORE = NCHUNK // NC
CPW = 80
CPW_LAST = NCHUNK - CPW * (NC * NS - 1)
NSLOT = 4
DRPT = N // NS
DEG_NWB = 10
DEG_WBR = N // DEG_NWB
DW = 16

_mesh = plsc.VectorSubcoreMesh(
    core_axis_name="c", subcore_axis_name="s", num_cores=NC, num_subcores=NS)


def _zero_vmem_2d(ref, rows, width):
  z16 = jnp.zeros((16,), jnp.float32)

  def row(i, _):
    for k in range(width // 16):
      ref[i, pl.ds(k * 16, 16)] = z16
    return 0

  lax.fori_loop(0, rows, row, 0)


def _zero_spmem_rows(acc, zb, base, nrows):
  zrows = zb.shape[0]
  full = nrows // zrows
  rem = nrows - full * zrows
  for k in range(full):
    pltpu.sync_copy(zb, acc.at[pl.ds(base + k * zrows, zrows)])
  if rem:
    pltpu.sync_copy(zb.at[pl.ds(0, rem)],
                    acc.at[pl.ds(base + full * zrows, rem)])


@functools.partial(
    pl.kernel,
    out_type=jax.ShapeDtypeStruct((NC, N, DW), jnp.float32),
    mesh=_mesh,
    compiler_params=pltpu.CompilerParams(use_tc_tiling_on_sc=False),
    scratch_types=[
        pltpu.VMEM_SHARED((N, DW), jnp.float32),
        pltpu.VMEM((C, DW), jnp.float32),
        pltpu.VMEM((C, DW), jnp.float32),
        pltpu.VMEM((NSLOT, C), jnp.int32),
        pltpu.SemaphoreType.DMA,
        pltpu.SemaphoreType.DMA,
        pltpu.SemaphoreType.DMA,
        pltpu.SemaphoreType.DMA,
        pltpu.SemaphoreType.DMA,
        pltpu.SemaphoreType.DMA,
        pltpu.SemaphoreType.DMA,
        pltpu.SemaphoreType.DMA,
    ],
)
def _deg_kernel(dst2d, out, degacc, ones, zb, didx,
                i0, i1, i2, i3, s0, s1, s2, s3):
  c = lax.axis_index("c")
  s = lax.axis_index("s")
  w = c * NS + s
  start = w * CPW
  cnt = jnp.where(w == NC * NS - 1, CPW_LAST, CPW)

  isems = (i0, i1, i2, i3)
  ssems = (s0, s1, s2, s3)

  def istart(j, k):
    pltpu.async_copy(dst2d.at[start + j], didx.at[k], isems[k])

  def iwait(j, k):
    pltpu.make_async_copy(dst2d.at[start + j], didx.at[k], isems[k]).wait()

  def sstart(k):
    pltpu.async_copy(ones, degacc.at[didx.at[k]], ssems[k], add=True)

  def swait(k):
    pltpu.make_async_copy(ones, degacc.at[didx.at[k]], ssems[k]).wait()

  for k in range(NSLOT):
    istart(k, k)

  one16 = jnp.ones((16,), jnp.float32)
  z16 = jnp.zeros((16,), jnp.float32)

  def fill(i, _):
    ones[i, pl.ds(0, 16)] = one16
    zb[i, pl.ds(0, 16)] = z16
    return 0

  lax.fori_loop(0, C, fill, 0)

  @pl.when(s < DEG_NWB)
  def _():
    _zero_spmem_rows(degacc, zb, s * DEG_WBR, DEG_WBR)

  plsc.subcore_barrier()

  def body(i, _):
    q = 4 * i
    for k in range(NSLOT):
      iwait(q + k, k)
      sstart(k)
    for k in range(NSLOT):
      @pl.when(q + k + NSLOT < cnt)
      def _():
        swait(k)
        istart(q + k + NSLOT, k)
    return 0

  lax.fori_loop(0, cnt // 4, body, 0)
  for k in range(NSLOT):
    swait(k)

  plsc.subcore_barrier()

  @pl.when(s < DEG_NWB)
  def _():
    pltpu.sync_copy(degacc.at[pl.ds(s * DEG_WBR, DEG_WBR)],
                    out.at[c, pl.ds(s * DEG_WBR, DEG_WBR)])


NBUF = 3
GSLOT = 6


@functools.partial(
    pl.kernel,
    out_type=jax.ShapeDtypeStruct((NC, N, D), jnp.float32),
    mesh=_mesh,
    compiler_params=pltpu.CompilerParams(use_tc_tiling_on_sc=False),
    scratch_types=[
        pltpu.VMEM_SHARED((N, D), jnp.float32),
        pltpu.VMEM((NBUF, C, D), jnp.float32),
        pltpu.VMEM((GSLOT, C), jnp.int32),
        pltpu.VMEM((GSLOT, C), jnp.int32),
        pltpu.SemaphoreType.DMA,
        pltpu.SemaphoreType.DMA,
        pltpu.SemaphoreType.DMA,
        pltpu.SemaphoreType.DMA,
        pltpu.SemaphoreType.DMA,
        pltpu.SemaphoreType.DMA,
        pltpu.SemaphoreType.DMA,
        pltpu.SemaphoreType.DMA,
        pltpu.SemaphoreType.DMA,
    ],
)
def _scatter_kernel(g, src2d, dst2d, out, acc, rows, sidx, didx,
                    i0, i1, i2, i3, i4, i5, g0, g1, g2):
  c = lax.axis_index("c")
  s = lax.axis_index("s")
  w = c * NS + s
  start = w * CPW
  cnt = jnp.where(w == NC * NS - 1, CPW_LAST, CPW)

  isems = (i0, i1, i2, i3, i4, i5)
  gsems = (g0, g1, g2)

  def istart(j, k):
    pltpu.async_copy(src2d.at[start + j], sidx.at[k], isems[k])
    pltpu.async_copy(dst2d.at[start + j], didx.at[k], isems[k])

  def iwait(j, k):
    pltpu.make_async_copy(src2d.at[start + j], sidx.at[k], isems[k]).wait()
    pltpu.make_async_copy(dst2d.at[start + j], didx.at[k], isems[k]).wait()

  def gstart(k, b):
    pltpu.async_copy(g.at[sidx.at[k]], rows.at[b], gsems[b])

  def gwait(k, b):
    pltpu.make_async_copy(g.at[sidx.at[k]], rows.at[b], gsems[b]).wait()

  def ssync(k, b):
    pltpu.sync_copy(rows.at[b], acc.at[didx.at[k]], add=True)

  for k in range(GSLOT - 1):
    istart(k, k)

  z16 = jnp.zeros((16,), jnp.float32)

  def zrow(i, _):
    for k in range(D // 16):
      rows[0, i, pl.ds(k * 16, 16)] = z16
    return 0

  lax.fori_loop(0, C, zrow, 0)

  _zero_spmem_rows(acc, rows.at[0], s * DRPT, DRPT)

  plsc.subcore_barrier()

  iwait(0, 0)
  gstart(0, 0)
  iwait(1, 1)
  gstart(1, 1)

  def pos(j, k, b):
    @pl.when(j < cnt)
    def _():
      @pl.when(j + 2 < cnt)
      def _():
        iwait(j + 2, (k + 2) % GSLOT)
        gstart((k + 2) % GSLOT, (b + 2) % NBUF)

      gwait(k, b)
      ssync(k, b)

      @pl.when(j + 5 < cnt)
      def _():
        istart(j + 5, (k + 5) % GSLOT)

  def body(i, _):
    q = 6 * i
    for k in range(6):
      pos(q + k, k, k % 3)
    return 0

  lax.fori_loop(0, (cnt + 5) // 6, body, 0)

  plsc.subcore_barrier()

  pltpu.sync_copy(acc.at[pl.ds(s * DRPT, DRPT)],
                  out.at[c, pl.ds(s * DRPT, DRPT)])


_RB = 2000


def _dinv_block(da_ref, db_ref):
  deg = da_ref[:, 0:1] + db_ref[:, 0:1] + 1.0
  return lax.rsqrt(deg)


def _k1_body(x_ref, w_ref, da_ref, db_ref, o_ref):
  dinv = _dinv_block(da_ref, db_ref)
  h = jnp.dot(x_ref[...], w_ref[...], preferred_element_type=jnp.float32)
  o_ref[...] = h * dinv


def _k2_body(a0_ref, a1_ref, g_ref, b_ref, w_ref, da_ref, db_ref, o_ref):
  dinv = _dinv_block(da_ref, db_ref)
  t = (a0_ref[...] + a1_ref[...] + g_ref[...]) * dinv + b_ref[...]
  z = jnp.maximum(t, 0.0)
  h = jnp.dot(z, w_ref[...], preferred_element_type=jnp.float32)
  o_ref[...] = h * dinv


def _k3_body(a0_ref, a1_ref, g_ref, b_ref, da_ref, db_ref, o_ref):
  dinv = _dinv_block(da_ref, db_ref)
  o_ref[...] = (a0_ref[...] + a1_ref[...] + g_ref[...]) * dinv + b_ref[...]


def _row_spec(width):
  return pl.BlockSpec((_RB, width), lambda i: (i, 0))


def _full_spec(shape):
  return pl.BlockSpec(shape, lambda i: (0,) * len(shape))


def _tc_call(body, in_specs, n_out_width=D):
  return pl.pallas_call(
      body,
      grid=(N // _RB,),
      in_specs=in_specs,
      out_specs=_row_spec(n_out_width),
      out_shape=jax.ShapeDtypeStruct((N, n_out_width), jnp.float32),
  )


def kernel(x, edge_index, W1, b1, W2, b2):
  src2d = edge_index[0].reshape(NCHUNK, C)
  dst2d = edge_index[1].reshape(NCHUNK, C)

  degp = _deg_kernel(dst2d)
  dega, degb = degp[0], degp[1]

  g1 = _tc_call(
      _k1_body,
      [_row_spec(D), _full_spec((D, D)), _row_spec(DW), _row_spec(DW)],
  )(x, W1, dega, degb)

  s1 = _scatter_kernel(g1, src2d, dst2d)

  b1r = b1.reshape(1, D)
  b2r = b2.reshape(1, D)

  g2 = _tc_call(
      _k2_body,
      [_row_spec(D), _row_spec(D), _row_spec(D), _full_spec((1, D)),
       _full_spec((D, D)), _row_spec(DW), _row_spec(DW)],
  )(s1[0], s1[1], g1, b1r, W2, dega, degb)

  s2 = _scatter_kernel(g2, src2d, dst2d)

  out = _tc_call(
      _k3_body,
      [_row_spec(D), _row_spec(D), _row_spec(D), _full_spec((1, D)),
       _row_spec(DW), _row_spec(DW)],
  )(s2[0], s2[1], g2, b2r, dega, degb)

  return out


# speedup vs baseline: 1.2442x; 1.0026x over previous
import functools

import jax
import jax.numpy as jnp
from jax import lax
from jax.experimental import pallas as pl
from jax.experimental.pallas import tpu as pltpu
from jax.experimental.pallas import tpu_sc as plsc

N = 10000
E = 320000
D = 128

NC = 2
NS = 16
C = 128
NCHUNK = E // C
CPW = 80
CPW_LAST = NCHUNK - CPW * (NC * NS - 1)
NSLOT = 4
DRPT = N // NS
DEG_NWB = 10
DEG_WBR = N // DEG_NWB
DW = 16

_mesh = plsc.VectorSubcoreMesh(
    core_axis_name="c", subcore_axis_name="s", num_cores=NC, num_subcores=NS)


def _zero_spmem_rows(acc, zb, base, nrows):
  zrows = zb.shape[0]
  full = nrows // zrows
  rem = nrows - full * zrows
  for k in range(full):
    pltpu.sync_copy(zb, acc.at[pl.ds(base + k * zrows, zrows)])
  if rem:
    pltpu.sync_copy(zb.at[pl.ds(0, rem)],
                    acc.at[pl.ds(base + full * zrows, rem)])


@functools.partial(
    pl.kernel,
    out_type=jax.ShapeDtypeStruct((NC, N, DW), jnp.float32),
    mesh=_mesh,
    compiler_params=pltpu.CompilerParams(use_tc_tiling_on_sc=False),
    scratch_types=[
        pltpu.VMEM_SHARED((N, DW), jnp.float32),
        pltpu.VMEM((C, DW), jnp.float32),
        pltpu.VMEM((C, DW), jnp.float32),
        pltpu.VMEM((NSLOT, C), jnp.int32),
        pltpu.SemaphoreType.DMA,
        pltpu.SemaphoreType.DMA,
        pltpu.SemaphoreType.DMA,
        pltpu.SemaphoreType.DMA,
        pltpu.SemaphoreType.DMA,
        pltpu.SemaphoreType.DMA,
        pltpu.SemaphoreType.DMA,
        pltpu.SemaphoreType.DMA,
    ],
)
def _deg_kernel(dst2d, out, degacc, ones, zb, didx,
                i0, i1, i2, i3, s0, s1, s2, s3):
  c = lax.axis_index("c")
  s = lax.axis_index("s")
  w = c * NS + s
  start = w * CPW
  cnt = jnp.where(w == NC * NS - 1, CPW_LAST, CPW)

  isems = (i0, i1, i2, i3)
  ssems = (s0, s1, s2, s3)

  def istart(j, k):
    pltpu.async_copy(dst2d.at[start + j], didx.at[k], isems[k])

  def iwait(j, k):
    pltpu.make_async_copy(dst2d.at[start + j], didx.at[k], isems[k]).wait()

  def sstart(k):
    pltpu.async_copy(ones, degacc.at[didx.at[k]], ssems[k], add=True)

  def swait(k):
    pltpu.make_async_copy(ones, degacc.at[didx.at[k]], ssems[k]).wait()

  for k in range(NSLOT):
    istart(k, k)

  one16 = jnp.ones((16,), jnp.float32)
  z16 = jnp.zeros((16,), jnp.float32)

  def fill(i, _):
    ones[i, pl.ds(0, 16)] = one16
    zb[i, pl.ds(0, 16)] = z16
    return 0

  lax.fori_loop(0, C, fill, 0)

  @pl.when(s < DEG_NWB)
  def _():
    _zero_spmem_rows(degacc, zb, s * DEG_WBR, DEG_WBR)

  plsc.subcore_barrier()

  def body(i, _):
    q = 4 * i
    for k in range(NSLOT):
      iwait(q + k, k)
      sstart(k)
    for k in range(NSLOT):
      @pl.when(q + k + NSLOT < cnt)
      def _():
        swait(k)
        istart(q + k + NSLOT, k)
    return 0

  lax.fori_loop(0, cnt // 4, body, 0)
  for k in range(NSLOT):
    swait(k)

  plsc.subcore_barrier()

  @pl.when(s < DEG_NWB)
  def _():
    pltpu.sync_copy(degacc.at[pl.ds(s * DEG_WBR, DEG_WBR)],
                    out.at[c, pl.ds(s * DEG_WBR, DEG_WBR)])


NBUF = 3
GSLOT = 6


@functools.partial(
    pl.kernel,
    out_type=jax.ShapeDtypeStruct((NC, N, D), jnp.float32),
    mesh=_mesh,
    compiler_params=pltpu.CompilerParams(use_tc_tiling_on_sc=False),
    scratch_types=[
        pltpu.VMEM_SHARED((N, D), jnp.float32),
        pltpu.VMEM((NBUF, C, D), jnp.float32),
        pltpu.VMEM((GSLOT, C), jnp.int32),
        pltpu.VMEM((GSLOT, C), jnp.int32),
        pltpu.SemaphoreType.DMA,
        pltpu.SemaphoreType.DMA,
        pltpu.SemaphoreType.DMA,
        pltpu.SemaphoreType.DMA,
        pltpu.SemaphoreType.DMA,
        pltpu.SemaphoreType.DMA,
        pltpu.SemaphoreType.DMA,
        pltpu.SemaphoreType.DMA,
        pltpu.SemaphoreType.DMA,
    ],
)
def _scatter_kernel(g, src2d, dst2d, out, acc, rows, sidx, didx,
                    i0, i1, i2, i3, i4, i5, g0, g1, g2):
  c = lax.axis_index("c")
  s = lax.axis_index("s")
  w = c * NS + s
  start = w * CPW
  cnt = jnp.where(w == NC * NS - 1, CPW_LAST, CPW)

  isems = (i0, i1, i2, i3, i4, i5)
  gsems = (g0, g1, g2)

  def istart(j, k):
    pltpu.async_copy(src2d.at[start + j], sidx.at[k], isems[k])
    pltpu.async_copy(dst2d.at[start + j], didx.at[k], isems[k])

  def iwait(j, k):
    pltpu.make_async_copy(src2d.at[start + j], sidx.at[k], isems[k]).wait()
    pltpu.make_async_copy(dst2d.at[start + j], didx.at[k], isems[k]).wait()

  def gstart(k, b):
    pltpu.async_copy(g.at[sidx.at[k]], rows.at[b], gsems[b])

  def gwait(k, b):
    pltpu.make_async_copy(g.at[sidx.at[k]], rows.at[b], gsems[b]).wait()

  def ssync(k, b):
    pltpu.sync_copy(rows.at[b], acc.at[didx.at[k]], add=True)

  for k in range(GSLOT - 1):
    istart(k, k)

  z16 = jnp.zeros((16,), jnp.float32)

  def zrow(i, _):
    for k in range(D // 16):
      rows[0, i, pl.ds(k * 16, 16)] = z16
    return 0

  lax.fori_loop(0, C, zrow, 0)

  _zero_spmem_rows(acc, rows.at[0], s * DRPT, DRPT)

  plsc.subcore_barrier()

  iwait(0, 0)
  gstart(0, 0)
  iwait(1, 1)
  gstart(1, 1)

  def pos(j, k, b):
    @pl.when(j < cnt)
    def _():
      @pl.when(j + 2 < cnt)
      def _():
        iwait(j + 2, (k + 2) % GSLOT)
        gstart((k + 2) % GSLOT, (b + 2) % NBUF)

      gwait(k, b)
      ssync(k, b)

      @pl.when(j + 5 < cnt)
      def _():
        istart(j + 5, (k + 5) % GSLOT)

  def body(i, _):
    q = 6 * i
    for k in range(6):
      pos(q + k, k, k % 3)
    return 0

  lax.fori_loop(0, (cnt + 5) // 6, body, 0)

  plsc.subcore_barrier()

  pltpu.sync_copy(acc.at[pl.ds(s * DRPT, DRPT)],
                  out.at[c, pl.ds(s * DRPT, DRPT)])


_RB = 2000


def _dinv_block(da_ref, db_ref):
  deg = da_ref[:, 0:1] + db_ref[:, 0:1] + 1.0
  return lax.rsqrt(deg)


def _k1_body(x_ref, w_ref, da_ref, db_ref, o_ref):
  dinv = _dinv_block(da_ref, db_ref)
  h = jnp.dot(x_ref[...], w_ref[...], preferred_element_type=jnp.float32)
  o_ref[...] = h * dinv


def _k2_body(a0_ref, a1_ref, g_ref, b_ref, w_ref, da_ref, db_ref, o_ref):
  dinv = _dinv_block(da_ref, db_ref)
  t = (a0_ref[...] + a1_ref[...] + g_ref[...]) * dinv + b_ref[...]
  z = jnp.maximum(t, 0.0)
  h = jnp.dot(z, w_ref[...], preferred_element_type=jnp.float32)
  o_ref[...] = h * dinv


def _k3_body(a0_ref, a1_ref, g_ref, b_ref, da_ref, db_ref, o_ref):
  dinv = _dinv_block(da_ref, db_ref)
  o_ref[...] = (a0_ref[...] + a1_ref[...] + g_ref[...]) * dinv + b_ref[...]


def _row_spec(width):
  return pl.BlockSpec((_RB, width), lambda i: (i, 0))


def _full_spec(shape):
  return pl.BlockSpec(shape, lambda i: (0,) * len(shape))


def _tc_call(body, in_specs, n_out_width=D):
  return pl.pallas_call(
      body,
      grid=(N // _RB,),
      in_specs=in_specs,
      out_specs=_row_spec(n_out_width),
      out_shape=jax.ShapeDtypeStruct((N, n_out_width), jnp.float32),
  )


def kernel(x, edge_index, W1, b1, W2, b2):
  src2d = edge_index[0].reshape(NCHUNK, C)
  dst2d = edge_index[1].reshape(NCHUNK, C)

  degp = _deg_kernel(dst2d)
  dega, degb = degp[0], degp[1]

  g1 = _tc_call(
      _k1_body,
      [_row_spec(D), _full_spec((D, D)), _row_spec(DW), _row_spec(DW)],
  )(x, W1, dega, degb)

  s1 = _scatter_kernel(g1, src2d, dst2d)

  b1r = b1.reshape(1, D)
  b2r = b2.reshape(1, D)

  g2 = _tc_call(
      _k2_body,
      [_row_spec(D), _row_spec(D), _row_spec(D), _full_spec((1, D)),
       _full_spec((D, D)), _row_spec(DW), _row_spec(DW)],
  )(s1[0], s1[1], g1, b1r, W2, dega, degb)

  s2 = _scatter_kernel(g2, src2d, dst2d)

  out = _tc_call(
      _k3_body,
      [_row_spec(D), _row_spec(D), _row_spec(D), _full_spec((1, D)),
       _row_spec(DW), _row_spec(DW)],
  )(s2[0], s2[1], g2, b2r, dega, degb)

  return out
